# baseline (speedup 1.0000x reference)
import functools

import jax
import jax.numpy as jnp
from jax import lax
from jax.experimental import pallas as pl
from jax.experimental.pallas import tpu as pltpu
from jax.experimental.pallas import tpu_sc as plsc

_N = 262144
_R = 1024
_SUM_H = 112
_SUM_W = 109
_E = 1024
_P_COLS = 384
_NW = 32

_LEVELS = (
    (64, 0, 0, 0, 0, 128, 256),
    (32, 63, 64, 64, 64, 192, 320),
    (16, 94, 96, 96, 96, 224, 352),
)


def _sc_gather_body(ht_hbm, wt_hbm, roi_hbm, uh_hbm, uw_hbm,
                    idx_v, hrows, wrows, sem_h, sem_w):
    bpw = _R // _NW
    wid = lax.axis_index("s") * 2 + lax.axis_index("c")
    base = wid * bpw
    pltpu.sync_copy(roi_hbm.at[pl.ds(base, bpw)], idx_v)

    copies = []
    for g in range(bpw // 16):
        vec = idx_v[pl.ds(g * 16, 16)]
        for j in range(16):
            i = g * 16 + j
            r = vec[j]
            copies.append(pltpu.async_copy(
                ht_hbm.at[pl.ds(r, 1)], hrows.at[pl.ds(i, 1)], sem_h))
            copies.append(pltpu.async_copy(
                wt_hbm.at[pl.ds(r, 1)], wrows.at[pl.ds(i, 1)], sem_w))
    for cp in copies:
        cp.wait()
    pltpu.sync_copy(hrows, uh_hbm.at[pl.ds(base, bpw)])
    pltpu.sync_copy(wrows, uw_hbm.at[pl.ds(base, bpw)])


def _sc_gather(ht, wt, roi):
    bpw = _R // _NW
    mesh = plsc.VectorSubcoreMesh(core_axis_name="c", subcore_axis_name="s")
    k = functools.partial(
        pl.kernel,
        mesh=mesh,
        out_type=[jax.ShapeDtypeStruct((_R, _SUM_H), jnp.float32),
                  jax.ShapeDtypeStruct((_R, _SUM_W), jnp.float32)],
        scratch_types=[pltpu.VMEM((bpw,), jnp.int32),
                       pltpu.VMEM((bpw, _SUM_H), jnp.float32),
                       pltpu.VMEM((bpw, _SUM_W), jnp.float32),
                       pltpu.SemaphoreType.DMA,
                       pltpu.SemaphoreType.DMA],
    )(_sc_gather_body)
    return k(ht, wt, roi)


def _precompute_body(uh_ref, uw_ref, p_ref):
    uh = uh_ref[...]
    uw = uw_ref[...]
    r = uh.shape[0]
    wsegs = []
    blsegs = []
    hsegs = []
    for (n, woff, hoff, _doff, _wc, _blc, _hc) in _LEVELS:
        uwl = uw[:, woff:woff + n - 1]
        m = jnp.max(uwl, axis=-1, keepdims=True)
        e = jnp.exp(uwl - m)
        w = e / jnp.sum(e, axis=-1, keepdims=True)
        tri = (lax.broadcasted_iota(jnp.int32, (n - 1, n - 1), 0)
               <= lax.broadcasted_iota(jnp.int32, (n - 1, n - 1), 1)
               ).astype(jnp.float32)
        cum = jnp.dot(w, tri, preferred_element_type=jnp.float32)
        bl = jnp.concatenate([jnp.zeros((r, 1), jnp.float32),
                              cum[:, :n - 2],
                              jnp.ones((r, 1), jnp.float32)], axis=1)
        h = jnp.exp(uh[:, hoff:hoff + n])
        wsegs.append(w)
        wsegs.append(jnp.zeros((r, {64: 1, 32: 1, 16: 17}[n]), jnp.float32))
        blsegs.append(bl)
        hsegs.append(h)
    z16 = jnp.zeros((r, 16), jnp.float32)
    p_ref[...] = jnp.concatenate(
        wsegs + blsegs + [z16] + hsegs + [z16], axis=1)


def _spline_body(x_ref, r_ref, d_ref, p_ref, out_ref, lad_ref, pa_ref):
    e_sz = x_ref.shape[2]
    xv = x_ref[0]
    rix = r_ref[0]
    r0 = r_ref[0, 0, 0]
    r1 = r_ref[0, 0, e_sz - 1]
    w0 = jnp.minimum((r0 // 8) * 8, _R - 32)
    iota_s = lax.broadcasted_iota(jnp.int32, (32, e_sz), 0)

    def win_dot(s_k, oh):
        return lax.dot_general(p_ref[pl.ds(s_k, 32), :], oh,
                               (((0,), (0,)), ((), ())),
                               preferred_element_type=jnp.float32)

    oh0 = (iota_s + w0 == rix).astype(jnp.float32)
    pa_ref[...] = win_dot(w0, oh0)
    for k in range(1, 32):
        thr = w0 + 32 * k

        @pl.when(r1 >= thr)
        def _():
            s_k = jnp.minimum(thr, _R - 32)
            ohk = ((iota_s + s_k == rix) & (rix >= thr)).astype(jnp.float32)
            pa_ref[...] = pa_ref[...] + win_dot(s_k, ohk)

    he_all = pa_ref[pl.ds(256, 112), :] * jnp.exp(d_ref[...])
    t_all = 0.5 * (he_all[:111] + he_all[1:]) * pa_ref[pl.ds(0, 111), :]
    lad = jnp.zeros((1, e_sz), jnp.float32)
    for (n, dof) in ((64, 0), (32, 64), (16, 96)):
        bl = pa_ref[pl.ds(128 + dof, n), :]
        wl = pa_ref[pl.ds(dof, n - 1), :]
        hel = he_all[dof:dof + n]
        tl = t_all[dof:dof + n - 1]
        cmpf = (xv >= bl).astype(jnp.float32)
        nxt = jnp.concatenate(
            [cmpf[1:n - 1], jnp.zeros((1, e_sz), jnp.float32)], axis=0)
        mcf = cmpf[:n - 1] - nxt

        def red(v):
            return jnp.sum(v, axis=0, keepdims=True)

        in_loc = red(mcf * bl[:n - 1])
        wq = red(mcf * wl)
        pc = red(nxt * tl)
        area = red(tl)
        hl_a = red(mcf * hel[:n - 1])
        hr_a = red(mcf * hel[1:n])
        inv_a = 1.0 / area
        h_l = hl_a * inv_a
        h_r = hr_a * inv_a
        in_cdf = pc * inv_a
        alpha = (xv - in_loc) / wq
        xv = 0.5 * (h_r - h_l) * wq * alpha * alpha + h_l * wq * alpha + in_cdf
        lad = lad + jnp.log(h_l + alpha * (h_r - h_l))
    out_ref[...] = xv.reshape(1, 1, e_sz)
    lad_ref[...] = lad.reshape(1, 1, e_sz)


def kernel(x, regions_oi, local_region_ix, delta, heights_table, widths_table):
    uh_all, uw_all = _sc_gather(heights_table, widths_table, regions_oi)
    p = pl.pallas_call(
        _precompute_body,
        out_shape=jax.ShapeDtypeStruct((_R, _P_COLS), jnp.float32),
    )(uh_all, uw_all)
    c = _N // _E
    x3 = x.reshape(c, 1, _E)
    r3 = local_region_ix.reshape(c, 1, _E)
    dt = delta.T
    out, lad = pl.pallas_call(
        _spline_body,
        grid=(c,),
        in_specs=[pl.BlockSpec((1, 1, _E), lambda i: (i, 0, 0)),
                  pl.BlockSpec((1, 1, _E), lambda i: (i, 0, 0)),
                  pl.BlockSpec((_SUM_H, _E), lambda i: (0, i)),
                  pl.BlockSpec((_R, _P_COLS), lambda i: (0, 0))],
        out_specs=[pl.BlockSpec((1, 1, _E), lambda i: (i, 0, 0)),
                   pl.BlockSpec((1, 1, _E), lambda i: (i, 0, 0))],
        out_shape=[jax.ShapeDtypeStruct((c, 1, _E), jnp.float32),
                   jax.ShapeDtypeStruct((c, 1, _E), jnp.float32)],
        scratch_shapes=[pltpu.VMEM((_P_COLS, _E), jnp.float32)],
    )(x3, r3, dt, p)
    return out.reshape(_N), lad.reshape(_N)


# speedup vs baseline: 1.0644x; 1.0644x over previous
import functools

import jax
import jax.numpy as jnp
from jax import lax
from jax.experimental import pallas as pl
from jax.experimental.pallas import tpu as pltpu
from jax.experimental.pallas import tpu_sc as plsc

_N = 262144
_R = 1024
_SUM_H = 112
_SUM_W = 109
_E = 1024
_P_COLS = 384
_NW = 32

_LEVELS = (
    (64, 0, 0, 0, 0, 128, 256),
    (32, 63, 64, 64, 64, 192, 320),
    (16, 94, 96, 96, 96, 224, 352),
)


def _sc_gather_body(ht_hbm, wt_hbm, roi_hbm, uh_hbm, uw_hbm,
                    idx_v, hrows, wrows, sem_h, sem_w):
    bpw = _R // _NW
    wid = lax.axis_index("s") * 2 + lax.axis_index("c")
    base = wid * bpw
    pltpu.sync_copy(roi_hbm.at[pl.ds(base, bpw)], idx_v)

    copies = []
    for g in range(bpw // 16):
        vec = idx_v[pl.ds(g * 16, 16)]
        for j in range(16):
            i = g * 16 + j
            r = vec[j]
            copies.append(pltpu.async_copy(
                ht_hbm.at[pl.ds(r, 1)], hrows.at[pl.ds(i, 1)], sem_h))
            copies.append(pltpu.async_copy(
                wt_hbm.at[pl.ds(r, 1)], wrows.at[pl.ds(i, 1)], sem_w))
    for cp in copies:
        cp.wait()
    pltpu.sync_copy(hrows, uh_hbm.at[pl.ds(base, bpw)])
    pltpu.sync_copy(wrows, uw_hbm.at[pl.ds(base, bpw)])


def _sc_gather(ht, wt, roi):
    bpw = _R // _NW
    mesh = plsc.VectorSubcoreMesh(core_axis_name="c", subcore_axis_name="s")
    k = functools.partial(
        pl.kernel,
        mesh=mesh,
        out_type=[jax.ShapeDtypeStruct((_R, _SUM_H), jnp.float32),
                  jax.ShapeDtypeStruct((_R, _SUM_W), jnp.float32)],
        scratch_types=[pltpu.VMEM((bpw,), jnp.int32),
                       pltpu.VMEM((bpw, _SUM_H), jnp.float32),
                       pltpu.VMEM((bpw, _SUM_W), jnp.float32),
                       pltpu.SemaphoreType.DMA,
                       pltpu.SemaphoreType.DMA],
    )(_sc_gather_body)
    return k(ht, wt, roi)


def _precompute_body(uh_ref, uw_ref, p_ref):
    uh = uh_ref[...]
    uw = uw_ref[...]
    r = uh.shape[0]
    wsegs = []
    blsegs = []
    hsegs = []
    for (n, woff, hoff, _doff, _wc, _blc, _hc) in _LEVELS:
        uwl = uw[:, woff:woff + n - 1]
        m = jnp.max(uwl, axis=-1, keepdims=True)
        e = jnp.exp(uwl - m)
        w = e / jnp.sum(e, axis=-1, keepdims=True)
        tri = (lax.broadcasted_iota(jnp.int32, (n - 1, n - 1), 0)
               <= lax.broadcasted_iota(jnp.int32, (n - 1, n - 1), 1)
               ).astype(jnp.float32)
        cum = jnp.dot(w, tri, preferred_element_type=jnp.float32)
        bl = jnp.concatenate([jnp.zeros((r, 1), jnp.float32),
                              cum[:, :n - 2],
                              jnp.ones((r, 1), jnp.float32)], axis=1)
        h = jnp.exp(uh[:, hoff:hoff + n])
        wsegs.append(w)
        wsegs.append(jnp.zeros((r, {64: 1, 32: 1, 16: 17}[n]), jnp.float32))
        blsegs.append(bl)
        hsegs.append(h)
    z16 = jnp.zeros((r, 16), jnp.float32)
    p_ref[...] = jnp.concatenate(
        wsegs + blsegs + [z16] + hsegs + [z16], axis=1)


def _spline_body(x_ref, r_ref, d_ref, p_ref, out_ref, lad_ref, pa_ref):
    e_sz = x_ref.shape[2]
    xv = x_ref[0]
    rix = r_ref[0]
    r0 = r_ref[0, 0, 0]
    r1 = r_ref[0, 0, e_sz - 1]
    w0 = jnp.minimum((r0 // 8) * 8, _R - 64)
    iota_s = lax.broadcasted_iota(jnp.int32, (64, e_sz), 0)

    def win_dot(s_k, oh):
        return lax.dot_general(p_ref[pl.ds(s_k, 64), :], oh,
                               (((0,), (0,)), ((), ())),
                               preferred_element_type=jnp.float32)

    oh0 = (iota_s + w0 == rix).astype(jnp.float32)
    pa_ref[...] = win_dot(w0, oh0)
    for k in range(1, 16):
        thr = w0 + 64 * k

        @pl.when(r1 >= thr)
        def _():
            s_k = jnp.minimum(thr, _R - 64)
            ohk = ((iota_s + s_k == rix) & (rix >= thr)).astype(jnp.float32)
            pa_ref[...] = pa_ref[...] + win_dot(s_k, ohk)

    he_all = pa_ref[pl.ds(256, 112), :] * jnp.exp(d_ref[...])
    t_all = 0.5 * (he_all[:111] + he_all[1:]) * pa_ref[pl.ds(0, 111), :]
    lad = jnp.zeros((1, e_sz), jnp.float32)
    for (n, dof) in ((64, 0), (32, 64), (16, 96)):
        bl = pa_ref[pl.ds(128 + dof, n), :]
        wl = pa_ref[pl.ds(dof, n - 1), :]
        hel = he_all[dof:dof + n]
        tl = t_all[dof:dof + n - 1]
        cmpf = (xv >= bl).astype(jnp.float32)
        nxt = jnp.concatenate(
            [cmpf[1:n - 1], jnp.zeros((1, e_sz), jnp.float32)], axis=0)
        mcf = cmpf[:n - 1] - nxt

        def red(v):
            return jnp.sum(v, axis=0, keepdims=True)

        in_loc = red(mcf * bl[:n - 1])
        wq = red(mcf * wl)
        pc = red(nxt * tl)
        area = red(tl)
        hl_a = red(mcf * hel[:n - 1])
        hr_a = red(mcf * hel[1:n])
        inv_a = 1.0 / area
        h_l = hl_a * inv_a
        h_r = hr_a * inv_a
        in_cdf = pc * inv_a
        alpha = (xv - in_loc) / wq
        xv = 0.5 * (h_r - h_l) * wq * alpha * alpha + h_l * wq * alpha + in_cdf
        lad = lad + jnp.log(h_l + alpha * (h_r - h_l))
    out_ref[...] = xv.reshape(1, 1, e_sz)
    lad_ref[...] = lad.reshape(1, 1, e_sz)


def kernel(x, regions_oi, local_region_ix, delta, heights_table, widths_table):
    uh_all, uw_all = _sc_gather(heights_table, widths_table, regions_oi)
    p = pl.pallas_call(
        _precompute_body,
        out_shape=jax.ShapeDtypeStruct((_R, _P_COLS), jnp.float32),
    )(uh_all, uw_all)
    c = _N // _E
    x3 = x.reshape(c, 1, _E)
    r3 = local_region_ix.reshape(c, 1, _E)
    dt = delta.T
    out, lad = pl.pallas_call(
        _spline_body,
        grid=(c,),
        in_specs=[pl.BlockSpec((1, 1, _E), lambda i: (i, 0, 0)),
                  pl.BlockSpec((1, 1, _E), lambda i: (i, 0, 0)),
                  pl.BlockSpec((_SUM_H, _E), lambda i: (0, i)),
                  pl.BlockSpec((_R, _P_COLS), lambda i: (0, 0))],
        out_specs=[pl.BlockSpec((1, 1, _E), lambda i: (i, 0, 0)),
                   pl.BlockSpec((1, 1, _E), lambda i: (i, 0, 0))],
        out_shape=[jax.ShapeDtypeStruct((c, 1, _E), jnp.float32),
                   jax.ShapeDtypeStruct((c, 1, _E), jnp.float32)],
        scratch_shapes=[pltpu.VMEM((_P_COLS, _E), jnp.float32)],
    )(x3, r3, dt, p)
    return out.reshape(_N), lad.reshape(_N)
